# baseline (speedup 1.0000x reference)
import functools

import jax
import jax.numpy as jnp
from jax import lax
from jax.experimental import pallas as pl
from jax.experimental.pallas import tpu as pltpu
from jax.experimental.pallas import tpu_sc as plsc

N_NODES = 10000
N_EDGES = 320000
D = 128

NC = 2
NS = 16
CHUNK = 128
NBUF = 2
R0 = 40
R1 = 120
SEG = 8
GROUPS_PER_SEG = SEG // NBUF
TOTAL_ROWS = NS * (R0 + R1)
E_PAD = TOTAL_ROWS * CHUNK
N_PAD = 10240
ROWS_PER_TILE_OUT = N_PAD // NS


def _sc_aggregate_body(x_hbm, src_hbm, dst_hbm, w_hbm, out_hbm,
                       src_v, dst_v, w_v, r0, r1, acc_sh, g0, g1, s0, s1):
    c = lax.axis_index("c")
    s = lax.axis_index("s")
    rows = [r0, r1]
    gsem = [g0, g1]
    ssem = [s0, s1]

    zeros16 = jnp.zeros((16,), jnp.float32)

    def _zero_row(r, carry):
        for k in range(D // 16):
            r0[r, pl.ds(k * 16, 16)] = zeros16
        return carry
    lax.fori_loop(0, CHUNK, _zero_row, 0)
    for t in range(ROWS_PER_TILE_OUT // CHUNK):
        pltpu.sync_copy(
            r0, acc_sh.at[pl.ds(s * ROWS_PER_TILE_OUT + t * CHUNK, CHUNK)])

    for b in range(NBUF):
        pltpu.async_copy(x_hbm.at[pl.ds(0, CHUNK)], rows[b], ssem[b])
    plsc.subcore_barrier()

    is0 = c == 0
    row0 = jnp.where(is0, s * R0, NS * R0 + s * R1)
    n_seg = jnp.where(is0, R0 // SEG, R1 // SEG)

    def _drain(buf, sem):
        pltpu.make_async_copy(x_hbm.at[src_v.at[0]], buf, sem).wait()

    def _scale(j, rv):
        def _g(g, carry):
            wv = w_v[j, pl.ds(g * 16, 16)]
            for i in range(16):
                w = wv[i]
                for k in range(D // 16):
                    sl = pl.ds(k * 16, 16)
                    rv[g * 16 + i, sl] = rv[g * 16 + i, sl] * w
            return carry
        lax.fori_loop(0, CHUNK // 16, _g, 0)

    def _seg(seg, carry):
        for b in range(NBUF):
            _drain(rows[b], ssem[b])
        base = row0 + seg * SEG
        pltpu.sync_copy(src_hbm.at[pl.ds(base, SEG)], src_v)
        pltpu.sync_copy(dst_hbm.at[pl.ds(base, SEG)], dst_v)
        pltpu.sync_copy(w_hbm.at[pl.ds(base, SEG)], w_v)

        for b in range(NBUF):
            pltpu.async_copy(x_hbm.at[src_v.at[b]], rows[b], gsem[b])
        for b in range(NBUF):
            _drain(rows[b], gsem[b])
            _scale(b, rows[b])
            pltpu.async_copy(rows[b], acc_sh.at[dst_v.at[b]], ssem[b], add=True)

        def _group(t, inner):
            for b in range(NBUF):
                j = t * NBUF + b
                _drain(rows[b], ssem[b])
                pltpu.async_copy(x_hbm.at[src_v.at[j]], rows[b], gsem[b])
            for b in range(NBUF):
                j = t * NBUF + b
                _drain(rows[b], gsem[b])
                _scale(j, rows[b])
                pltpu.async_copy(rows[b], acc_sh.at[dst_v.at[j]], ssem[b],
                                 add=True)
            return inner
        lax.fori_loop(1, GROUPS_PER_SEG, _group, 0)
        return carry
    lax.fori_loop(0, n_seg, _seg, 0)

    for b in range(NBUF):
        _drain(rows[b], ssem[b])

    plsc.subcore_barrier()
    a0 = s * ROWS_PER_TILE_OUT
    pltpu.sync_copy(acc_sh.at[pl.ds(a0, ROWS_PER_TILE_OUT)],
                    out_hbm.at[c, pl.ds(a0, ROWS_PER_TILE_OUT)])


@functools.cache
def _sc_aggregate():
    return pl.kernel(
        _sc_aggregate_body,
        out_type=jax.ShapeDtypeStruct((NC, N_PAD, D), jnp.float32),
        mesh=plsc.VectorSubcoreMesh(core_axis_name="c", subcore_axis_name="s"),
        scratch_types=[
            pltpu.VMEM((SEG, CHUNK), jnp.int32),
            pltpu.VMEM((SEG, CHUNK), jnp.int32),
            pltpu.VMEM((SEG, CHUNK), jnp.float32),
        ] + [pltpu.VMEM((CHUNK, D), jnp.float32)] * NBUF
        + [pltpu.VMEM_SHARED((N_PAD, D), jnp.float32)]
        + [pltpu.SemaphoreType.DMA] * (2 * NBUF),
    )


def _tc_combine_body(a_ref, x_ref, wn_ref, ws_ref, b_ref, o_ref):
    a = a_ref[0] + a_ref[1]
    dn = (((1,), (1,)), ((), ()))
    o_ref[...] = (
        lax.dot_general(a, wn_ref[...], dn, preferred_element_type=jnp.float32)
        + lax.dot_general(x_ref[...], ws_ref[...], dn,
                          preferred_element_type=jnp.float32)
        + b_ref[...]
    )


def _tc_combine(a, x_pad, w_nei, w_self, b_self):
    blk = 1024
    grid = (N_PAD // blk,)
    return pl.pallas_call(
        _tc_combine_body,
        grid=grid,
        in_specs=[
            pl.BlockSpec((NC, blk, D), lambda i: (0, i, 0)),
            pl.BlockSpec((blk, D), lambda i: (i, 0)),
            pl.BlockSpec((D, D), lambda i: (0, 0)),
            pl.BlockSpec((D, D), lambda i: (0, 0)),
            pl.BlockSpec((1, D), lambda i: (0, 0)),
        ],
        out_specs=pl.BlockSpec((blk, D), lambda i: (i, 0)),
        out_shape=jax.ShapeDtypeStruct((N_PAD, D), jnp.float32),
    )(a, x_pad, w_nei, w_self, b_self)


def kernel(x, edge_index, edge_weight, W_self, b_self, W_nei):
    src = edge_index[0].astype(jnp.int32)
    dst = edge_index[1].astype(jnp.int32)
    w = edge_weight.astype(jnp.float32)

    pad = E_PAD - N_EDGES
    src_p = jnp.concatenate([src, jnp.zeros((pad,), jnp.int32)]).reshape(-1, CHUNK)
    dst_p = jnp.concatenate([dst, jnp.zeros((pad,), jnp.int32)]).reshape(-1, CHUNK)
    w_p = jnp.concatenate([w, jnp.zeros((pad,), jnp.float32)]).reshape(-1, CHUNK)

    a = _sc_aggregate()(x, src_p, dst_p, w_p)

    x_pad = jnp.concatenate([x, jnp.zeros((N_PAD - N_NODES, D), jnp.float32)])
    out = _tc_combine(a, x_pad, W_nei, W_self, b_self.reshape(1, D))
    return out[:N_NODES]


# speedup vs baseline: 1.2959x; 1.2959x over previous
import functools

import jax
import jax.numpy as jnp
from jax import lax
from jax.experimental import pallas as pl
from jax.experimental.pallas import tpu as pltpu
from jax.experimental.pallas import tpu_sc as plsc

N_NODES = 10000
N_EDGES = 320000
D = 128

NC = 2
NS = 16
CHUNK = 128
NBUF = 2
R0 = 120
R1 = 40
SEG = 8
GROUPS_PER_SEG = SEG // NBUF
TOTAL_ROWS = NS * (R0 + R1)
E_PAD = TOTAL_ROWS * CHUNK
N_PAD = 10240
ROWS_PER_TILE_OUT = N_PAD // NS


def _sc_aggregate_body(x_hbm, src_hbm, dst_hbm, w_hbm, out_hbm,
                       src_v, dst_v, w_v, r0, r1, acc_sh, g0, g1, s0, s1):
    c = lax.axis_index("c")
    s = lax.axis_index("s")
    rows = [r0, r1]
    gsem = [g0, g1]
    ssem = [s0, s1]

    zeros16 = jnp.zeros((16,), jnp.float32)

    def _zero_row(r, carry):
        for k in range(D // 16):
            r0[r, pl.ds(k * 16, 16)] = zeros16
        return carry
    lax.fori_loop(0, CHUNK, _zero_row, 0)
    for t in range(ROWS_PER_TILE_OUT // CHUNK):
        pltpu.sync_copy(
            r0, acc_sh.at[pl.ds(s * ROWS_PER_TILE_OUT + t * CHUNK, CHUNK)])

    for b in range(NBUF):
        pltpu.async_copy(x_hbm.at[pl.ds(0, CHUNK)], rows[b], ssem[b])
    plsc.subcore_barrier()

    is0 = c == 0
    row0 = jnp.where(is0, s * R0, NS * R0 + s * R1)
    n_seg = jnp.where(is0, R0 // SEG, R1 // SEG)

    def _drain(buf, sem):
        pltpu.make_async_copy(x_hbm.at[src_v.at[0]], buf, sem).wait()

    def _scale(j, rv):
        def _g(g, carry):
            wv = w_v[j, pl.ds(g * 16, 16)]
            for i in range(16):
                w = wv[i]
                for k in range(D // 16):
                    sl = pl.ds(k * 16, 16)
                    rv[g * 16 + i, sl] = rv[g * 16 + i, sl] * w
            return carry
        lax.fori_loop(0, CHUNK // 16, _g, 0)

    def _seg(seg, carry):
        for b in range(NBUF):
            _drain(rows[b], ssem[b])
        base = row0 + seg * SEG
        pltpu.sync_copy(src_hbm.at[pl.ds(base, SEG)], src_v)
        pltpu.sync_copy(dst_hbm.at[pl.ds(base, SEG)], dst_v)
        pltpu.sync_copy(w_hbm.at[pl.ds(base, SEG)], w_v)

        for b in range(NBUF):
            pltpu.async_copy(x_hbm.at[src_v.at[b]], rows[b], gsem[b])
        for b in range(NBUF):
            _drain(rows[b], gsem[b])
            _scale(b, rows[b])
            pltpu.async_copy(rows[b], acc_sh.at[dst_v.at[b]], ssem[b], add=True)

        def _group(t, inner):
            for b in range(NBUF):
                j = t * NBUF + b
                _drain(rows[b], ssem[b])
                pltpu.async_copy(x_hbm.at[src_v.at[j]], rows[b], gsem[b])
            for b in range(NBUF):
                j = t * NBUF + b
                _drain(rows[b], gsem[b])
                _scale(j, rows[b])
                pltpu.async_copy(rows[b], acc_sh.at[dst_v.at[j]], ssem[b],
                                 add=True)
            return inner
        lax.fori_loop(1, GROUPS_PER_SEG, _group, 0)
        return carry
    lax.fori_loop(0, n_seg, _seg, 0)

    for b in range(NBUF):
        _drain(rows[b], ssem[b])

    plsc.subcore_barrier()
    a0 = s * ROWS_PER_TILE_OUT
    pltpu.sync_copy(acc_sh.at[pl.ds(a0, ROWS_PER_TILE_OUT)],
                    out_hbm.at[c, pl.ds(a0, ROWS_PER_TILE_OUT)])


@functools.cache
def _sc_aggregate():
    return pl.kernel(
        _sc_aggregate_body,
        out_type=jax.ShapeDtypeStruct((NC, N_PAD, D), jnp.float32),
        mesh=plsc.VectorSubcoreMesh(core_axis_name="c", subcore_axis_name="s"),
        scratch_types=[
            pltpu.VMEM((SEG, CHUNK), jnp.int32),
            pltpu.VMEM((SEG, CHUNK), jnp.int32),
            pltpu.VMEM((SEG, CHUNK), jnp.float32),
        ] + [pltpu.VMEM((CHUNK, D), jnp.float32)] * NBUF
        + [pltpu.VMEM_SHARED((N_PAD, D), jnp.float32)]
        + [pltpu.SemaphoreType.DMA] * (2 * NBUF),
    )


def _tc_combine_body(a_ref, x_ref, wn_ref, ws_ref, b_ref, o_ref):
    a = a_ref[0] + a_ref[1]
    dn = (((1,), (1,)), ((), ()))
    o_ref[...] = (
        lax.dot_general(a, wn_ref[...], dn, preferred_element_type=jnp.float32)
        + lax.dot_general(x_ref[...], ws_ref[...], dn,
                          preferred_element_type=jnp.float32)
        + b_ref[...]
    )


def _tc_combine(a, x_pad, w_nei, w_self, b_self):
    blk = 1024
    grid = (N_PAD // blk,)
    return pl.pallas_call(
        _tc_combine_body,
        grid=grid,
        in_specs=[
            pl.BlockSpec((NC, blk, D), lambda i: (0, i, 0)),
            pl.BlockSpec((blk, D), lambda i: (i, 0)),
            pl.BlockSpec((D, D), lambda i: (0, 0)),
            pl.BlockSpec((D, D), lambda i: (0, 0)),
            pl.BlockSpec((1, D), lambda i: (0, 0)),
        ],
        out_specs=pl.BlockSpec((blk, D), lambda i: (i, 0)),
        out_shape=jax.ShapeDtypeStruct((N_PAD, D), jnp.float32),
    )(a, x_pad, w_nei, w_self, b_self)


def kernel(x, edge_index, edge_weight, W_self, b_self, W_nei):
    src = edge_index[0].astype(jnp.int32)
    dst = edge_index[1].astype(jnp.int32)
    w = edge_weight.astype(jnp.float32)

    pad = E_PAD - N_EDGES
    src_p = jnp.concatenate([src, jnp.zeros((pad,), jnp.int32)]).reshape(-1, CHUNK)
    dst_p = jnp.concatenate([dst, jnp.zeros((pad,), jnp.int32)]).reshape(-1, CHUNK)
    w_p = jnp.concatenate([w, jnp.zeros((pad,), jnp.float32)]).reshape(-1, CHUNK)

    a = _sc_aggregate()(x, src_p, dst_p, w_p)

    x_pad = jnp.concatenate([x, jnp.zeros((N_PAD - N_NODES, D), jnp.float32)])
    out = _tc_combine(a, x_pad, W_nei, W_self, b_self.reshape(1, D))
    return out[:N_NODES]
